# baseline (device time: 169630 ns/iter reference)
import jax
import jax.numpy as jnp
from jax import lax
from jax.experimental import pallas as pl
from jax.experimental.pallas import tpu as pltpu

N_DEV = 4


def kernel(x, w_mat, scale_x, scale_w):
    m_per, k = x.shape
    _, n_per = w_mat.shape

    def body(x_ref, w_ref, sx_ref, sw_ref, out_ref,
             comm_ref, send_sems, recv_sems):
        my = lax.axis_index("i")
        left = lax.rem(my + (N_DEV - 1), N_DEV)
        right = lax.rem(my + 1, N_DEV)

        barrier_sem = pltpu.get_barrier_semaphore()
        for nbr in (left, right):
            pl.semaphore_signal(
                barrier_sem, inc=1,
                device_id=(nbr,), device_id_type=pl.DeviceIdType.MESH,
            )
        pl.semaphore_wait(barrier_sem, 2)

        scale = sx_ref[0] * sw_ref[0]

        comm_ref[0] = x_ref[...]
        acc = jnp.dot(x_ref[...], w_ref[...],
                      preferred_element_type=jnp.int32)
        out_ref[pl.ds(my * m_per, m_per), :] = acc.astype(jnp.float32) * scale

        for h in range(N_DEV - 1):
            send_slot = h % 2
            recv_slot = (h + 1) % 2
            rdma = pltpu.make_async_remote_copy(
                src_ref=comm_ref.at[send_slot],
                dst_ref=comm_ref.at[recv_slot],
                send_sem=send_sems.at[send_slot],
                recv_sem=recv_sems.at[recv_slot],
                device_id=(right,),
                device_id_type=pl.DeviceIdType.MESH,
            )
            rdma.start()
            rdma.wait()

            origin = lax.rem(my + (N_DEV - 1 - h), N_DEV)
            acc = jnp.dot(comm_ref[recv_slot], w_ref[...],
                          preferred_element_type=jnp.int32)
            out_ref[pl.ds(origin * m_per, m_per), :] = (
                acc.astype(jnp.float32) * scale
            )

    return pl.pallas_call(
        body,
        out_shape=jax.ShapeDtypeStruct((N_DEV * m_per, n_per), jnp.float32),
        in_specs=[
            pl.BlockSpec(memory_space=pltpu.VMEM),
            pl.BlockSpec(memory_space=pltpu.VMEM),
            pl.BlockSpec(memory_space=pltpu.VMEM),
            pl.BlockSpec(memory_space=pltpu.VMEM),
        ],
        out_specs=pl.BlockSpec(memory_space=pltpu.VMEM),
        scratch_shapes=[
            pltpu.VMEM((2, m_per, k), x.dtype),
            pltpu.SemaphoreType.DMA((2,)),
            pltpu.SemaphoreType.DMA((2,)),
        ],
        compiler_params=pltpu.CompilerParams(collective_id=0),
    )(x, w_mat, scale_x, scale_w)


# device time: 89018 ns/iter; 1.9056x vs baseline; 1.9056x over previous
import functools

import jax
import jax.numpy as jnp
from jax import lax
from jax.experimental import pallas as pl
from jax.experimental.pallas import tpu as pltpu

N_DEV = 4


def kernel(x, w_mat, scale_x, scale_w):
    m_per, k = x.shape
    _, n_per = w_mat.shape
    half = m_per // 2

    def body(x_ref, w_ref, sx_ref, sw_ref, out_ref,
             buf_l, buf_r, buf_l2, buf_r2, send_sems, recv_sems):
        my = lax.axis_index("i")
        left = lax.rem(my + (N_DEV - 1), N_DEV)
        right = lax.rem(my + 1, N_DEV)
        opp = lax.rem(my + 2, N_DEV)

        barrier_sem = pltpu.get_barrier_semaphore()
        for nbr in (left, right):
            pl.semaphore_signal(
                barrier_sem, inc=1,
                device_id=(nbr,), device_id_type=pl.DeviceIdType.MESH,
            )
        pl.semaphore_wait(barrier_sem, 2)

        send_r = pltpu.make_async_remote_copy(
            src_ref=x_ref, dst_ref=buf_l,
            send_sem=send_sems.at[0], recv_sem=recv_sems.at[0],
            device_id=(right,), device_id_type=pl.DeviceIdType.MESH,
        )
        send_l = pltpu.make_async_remote_copy(
            src_ref=x_ref, dst_ref=buf_r,
            send_sem=send_sems.at[1], recv_sem=recv_sems.at[1],
            device_id=(left,), device_id_type=pl.DeviceIdType.MESH,
        )
        send_r.start()
        send_l.start()

        scale = sx_ref[0] * sw_ref[0]

        acc = jnp.dot(x_ref[...], w_ref[...],
                      preferred_element_type=jnp.int32)
        out_ref[pl.ds(my * m_per, m_per), :] = acc.astype(jnp.float32) * scale

        send_r.wait_recv()
        fwd_r = pltpu.make_async_remote_copy(
            src_ref=buf_l.at[pl.ds(0, half)], dst_ref=buf_l2,
            send_sem=send_sems.at[2], recv_sem=recv_sems.at[2],
            device_id=(right,), device_id_type=pl.DeviceIdType.MESH,
        )
        fwd_r.start()
        acc = jnp.dot(buf_l[...], w_ref[...],
                      preferred_element_type=jnp.int32)
        out_ref[pl.ds(left * m_per, m_per), :] = (
            acc.astype(jnp.float32) * scale
        )

        send_l.wait_recv()
        fwd_l = pltpu.make_async_remote_copy(
            src_ref=buf_r.at[pl.ds(half, half)], dst_ref=buf_r2,
            send_sem=send_sems.at[3], recv_sem=recv_sems.at[3],
            device_id=(left,), device_id_type=pl.DeviceIdType.MESH,
        )
        fwd_l.start()
        acc = jnp.dot(buf_r[...], w_ref[...],
                      preferred_element_type=jnp.int32)
        out_ref[pl.ds(right * m_per, m_per), :] = (
            acc.astype(jnp.float32) * scale
        )

        fwd_r.wait_recv()
        acc = jnp.dot(buf_l2[...], w_ref[...],
                      preferred_element_type=jnp.int32)
        out_ref[pl.ds(opp * m_per, half), :] = acc.astype(jnp.float32) * scale

        fwd_l.wait_recv()
        acc = jnp.dot(buf_r2[...], w_ref[...],
                      preferred_element_type=jnp.int32)
        out_ref[pl.ds(opp * m_per + half, half), :] = (
            acc.astype(jnp.float32) * scale
        )

        send_r.wait_send()
        send_l.wait_send()
        fwd_r.wait_send()
        fwd_l.wait_send()

        @functools.partial(
            pl.run_scoped, second_barrier=pltpu.SemaphoreType.REGULAR
        )
        def _(second_barrier):
            for nbr in (left, right):
                pl.semaphore_signal(
                    second_barrier, inc=1,
                    device_id=(nbr,), device_id_type=pl.DeviceIdType.MESH,
                )
            pl.semaphore_wait(second_barrier, 2)

    return pl.pallas_call(
        body,
        out_shape=jax.ShapeDtypeStruct((N_DEV * m_per, n_per), jnp.float32),
        in_specs=[
            pl.BlockSpec(memory_space=pltpu.VMEM),
            pl.BlockSpec(memory_space=pltpu.VMEM),
            pl.BlockSpec(memory_space=pltpu.VMEM),
            pl.BlockSpec(memory_space=pltpu.VMEM),
        ],
        out_specs=pl.BlockSpec(memory_space=pltpu.VMEM),
        scratch_shapes=[
            pltpu.VMEM((m_per, k), x.dtype),
            pltpu.VMEM((m_per, k), x.dtype),
            pltpu.VMEM((half, k), x.dtype),
            pltpu.VMEM((half, k), x.dtype),
            pltpu.SemaphoreType.DMA((4,)),
            pltpu.SemaphoreType.DMA((4,)),
        ],
        compiler_params=pltpu.CompilerParams(collective_id=0),
    )(x, w_mat, scale_x, scale_w)


# device time: 85066 ns/iter; 1.9941x vs baseline; 1.0465x over previous
import functools

import jax
import jax.numpy as jnp
from jax import lax
from jax.experimental import pallas as pl
from jax.experimental.pallas import tpu as pltpu

N_DEV = 4


def kernel(x, w_mat, scale_x, scale_w):
    m_per, k = x.shape
    _, n_per = w_mat.shape
    half = m_per // 2

    def body(x_hbm, w_hbm, sx_ref, sw_ref, out_ref,
             x_vmem, w_vmem, buf_l, buf_r, buf_l2, buf_r2,
             local_sems, send_sems, recv_sems):
        my = lax.axis_index("i")
        left = lax.rem(my + (N_DEV - 1), N_DEV)
        right = lax.rem(my + 1, N_DEV)
        opp = lax.rem(my + 2, N_DEV)

        cp_x = pltpu.make_async_copy(x_hbm, x_vmem, local_sems.at[0])
        cp_w = pltpu.make_async_copy(w_hbm, w_vmem, local_sems.at[1])
        cp_x.start()
        cp_w.start()

        barrier_sem = pltpu.get_barrier_semaphore()
        for nbr in (left, right):
            pl.semaphore_signal(
                barrier_sem, inc=1,
                device_id=(nbr,), device_id_type=pl.DeviceIdType.MESH,
            )
        pl.semaphore_wait(barrier_sem, 2)

        send_r1 = pltpu.make_async_remote_copy(
            src_ref=x_hbm.at[pl.ds(0, half)],
            dst_ref=buf_l.at[pl.ds(0, half)],
            send_sem=send_sems.at[0], recv_sem=recv_sems.at[0],
            device_id=(right,), device_id_type=pl.DeviceIdType.MESH,
        )
        send_l1 = pltpu.make_async_remote_copy(
            src_ref=x_hbm.at[pl.ds(half, half)],
            dst_ref=buf_r.at[pl.ds(half, half)],
            send_sem=send_sems.at[1], recv_sem=recv_sems.at[1],
            device_id=(left,), device_id_type=pl.DeviceIdType.MESH,
        )
        send_r2 = pltpu.make_async_remote_copy(
            src_ref=x_hbm.at[pl.ds(half, half)],
            dst_ref=buf_l.at[pl.ds(half, half)],
            send_sem=send_sems.at[2], recv_sem=recv_sems.at[2],
            device_id=(right,), device_id_type=pl.DeviceIdType.MESH,
        )
        send_l2 = pltpu.make_async_remote_copy(
            src_ref=x_hbm.at[pl.ds(0, half)],
            dst_ref=buf_r.at[pl.ds(0, half)],
            send_sem=send_sems.at[3], recv_sem=recv_sems.at[3],
            device_id=(left,), device_id_type=pl.DeviceIdType.MESH,
        )
        send_r1.start()
        send_l1.start()
        send_r2.start()
        send_l2.start()

        scale = sx_ref[0] * sw_ref[0]

        cp_x.wait()
        cp_w.wait()
        acc = jnp.dot(x_vmem[...], w_vmem[...],
                      preferred_element_type=jnp.int32)
        out_ref[pl.ds(my * m_per, m_per), :] = acc.astype(jnp.float32) * scale

        send_r1.wait_recv()
        fwd_r = pltpu.make_async_remote_copy(
            src_ref=buf_l.at[pl.ds(0, half)], dst_ref=buf_l2,
            send_sem=send_sems.at[4], recv_sem=recv_sems.at[4],
            device_id=(right,), device_id_type=pl.DeviceIdType.MESH,
        )
        fwd_r.start()
        send_l1.wait_recv()
        fwd_l = pltpu.make_async_remote_copy(
            src_ref=buf_r.at[pl.ds(half, half)], dst_ref=buf_r2,
            send_sem=send_sems.at[5], recv_sem=recv_sems.at[5],
            device_id=(left,), device_id_type=pl.DeviceIdType.MESH,
        )
        fwd_l.start()

        send_r2.wait_recv()
        acc = jnp.dot(buf_l[...], w_vmem[...],
                      preferred_element_type=jnp.int32)
        out_ref[pl.ds(left * m_per, m_per), :] = (
            acc.astype(jnp.float32) * scale
        )
        send_l2.wait_recv()
        acc = jnp.dot(buf_r[...], w_vmem[...],
                      preferred_element_type=jnp.int32)
        out_ref[pl.ds(right * m_per, m_per), :] = (
            acc.astype(jnp.float32) * scale
        )

        fwd_r.wait_recv()
        acc = jnp.dot(buf_l2[...], w_vmem[...],
                      preferred_element_type=jnp.int32)
        out_ref[pl.ds(opp * m_per, half), :] = acc.astype(jnp.float32) * scale
        fwd_l.wait_recv()
        acc = jnp.dot(buf_r2[...], w_vmem[...],
                      preferred_element_type=jnp.int32)
        out_ref[pl.ds(opp * m_per + half, half), :] = (
            acc.astype(jnp.float32) * scale
        )

        send_r1.wait_send()
        send_l1.wait_send()
        send_r2.wait_send()
        send_l2.wait_send()
        fwd_r.wait_send()
        fwd_l.wait_send()

        @functools.partial(
            pl.run_scoped, second_barrier=pltpu.SemaphoreType.REGULAR
        )
        def _(second_barrier):
            for nbr in (left, right):
                pl.semaphore_signal(
                    second_barrier, inc=1,
                    device_id=(nbr,), device_id_type=pl.DeviceIdType.MESH,
                )
            pl.semaphore_wait(second_barrier, 2)

    return pl.pallas_call(
        body,
        out_shape=jax.ShapeDtypeStruct((N_DEV * m_per, n_per), jnp.float32),
        in_specs=[
            pl.BlockSpec(memory_space=pl.ANY),
            pl.BlockSpec(memory_space=pl.ANY),
            pl.BlockSpec(memory_space=pltpu.VMEM),
            pl.BlockSpec(memory_space=pltpu.VMEM),
        ],
        out_specs=pl.BlockSpec(memory_space=pltpu.VMEM),
        scratch_shapes=[
            pltpu.VMEM((m_per, k), x.dtype),
            pltpu.VMEM((k, n_per), w_mat.dtype),
            pltpu.VMEM((m_per, k), x.dtype),
            pltpu.VMEM((m_per, k), x.dtype),
            pltpu.VMEM((half, k), x.dtype),
            pltpu.VMEM((half, k), x.dtype),
            pltpu.SemaphoreType.DMA((2,)),
            pltpu.SemaphoreType.DMA((6,)),
            pltpu.SemaphoreType.DMA((6,)),
        ],
        compiler_params=pltpu.CompilerParams(collective_id=0),
    )(x, w_mat, scale_x, scale_w)
